# baseline (speedup 1.0000x reference)
import functools

import jax
import jax.numpy as jnp
from jax import lax
from jax.experimental import pallas as pl
from jax.experimental.pallas import tpu as pltpu
from jax.experimental.pallas import tpu_sc as plsc

NV = 100000
ED = 48
B = 16384
N_CONT = 4
N_ID = 13
X_D = 17
OUT_D = N_CONT + N_ID * ED
PAD_D = 640
NT = PAD_D // 128

NC = 2
NS = 16
NW = NC * NS
ROWS_W = B // NW
CHUNK = 64
N_CHUNK = ROWS_W // CHUNK
NRING = 4

_mesh = plsc.VectorSubcoreMesh(
    core_axis_name="c", subcore_axis_name="s", num_cores=NC, num_subcores=NS
)


@functools.partial(
    pl.kernel,
    out_type=jax.ShapeDtypeStruct((B * NT, 128), jnp.float32),
    mesh=_mesh,
    compiler_params=pltpu.CompilerParams(
        needs_layout_passes=False, use_tc_tiling_on_sc=False
    ),
    scratch_types=[
        pltpu.VMEM((ROWS_W, X_D), jnp.float32),
        pltpu.VMEM((NRING, CHUNK), jnp.int32),
        pltpu.VMEM((NRING, CHUNK, ED), jnp.float32),
        pltpu.VMEM((2, CHUNK * NT, 128), jnp.float32),
        [pltpu.SemaphoreType.DMA] * NRING,
        [pltpu.SemaphoreType.DMA] * 2,
    ],
)
def _card_emb(x_hbm, emb_hbm, out_hbm, x_v, idx_v, rows_v, outbuf_v, gsems, wsems):
    wid = lax.axis_index("s") * NC + lax.axis_index("c")
    base = wid * ROWS_W

    lane = lax.iota(jnp.int32, 16)

    pltpu.sync_copy(x_hbm.at[pl.ds(base, ROWS_W), :], x_v)

    def chunk_body(k, carry):
        p = lax.bitwise_and(k, 1)
        loc0 = k * CHUNK
        obuf = outbuf_v.at[p]

        for par in range(2):
            @pl.when(jnp.logical_and(k >= 2, p == par))
            def _(par=par):
                pltpu.make_async_copy(
                    obuf, out_hbm.at[pl.ds(base * NT, CHUNK * NT)], wsems[par]
                ).wait()

        def build_and_fire(j):
            slot = j % NRING
            idcol = jnp.full((16,), N_CONT + j, jnp.int32)
            for g in range(CHUNK // 16):
                vals = plsc.load_gather(x_v, [loc0 + g * 16 + lane, idcol])
                idx_v[slot, pl.ds(g * 16, 16)] = vals.astype(jnp.int32)
            pltpu.async_copy(
                emb_hbm.at[idx_v.at[slot]], rows_v.at[slot], gsems[slot]
            )

        for j in range(min(NRING - 1, N_ID)):
            build_and_fire(j)

        def cont_body(i, cc):
            t = lane + i * 16
            row = lax.shift_right_logical(t, 2)
            col = lax.bitwise_and(t, 3)
            vals = plsc.load_gather(x_v, [loc0 + row, col])
            plsc.store_scatter(obuf, [row * NT, col], vals)
            return cc

        lax.fori_loop(0, CHUNK * N_CONT // 16, cont_body, 0)

        for j in range(N_ID):
            if j + NRING - 1 < N_ID:
                build_and_fire(j + NRING - 1)
            slot = j % NRING
            pltpu.make_async_copy(
                emb_hbm.at[idx_v.at[slot]], rows_v.at[slot], gsems[slot]
            ).wait()
            rbuf = rows_v.at[slot]
            segs = []
            for m in range(ED // 16):
                c0 = N_CONT + j * ED + m * 16
                segs.append((m, c0 // 128, c0 % 128, c0 % 128 > 112))

            def copy_body(r, cc):
                r5 = r * NT
                for m, t, c, crosses in segs:
                    v = rbuf[r, pl.ds(m * 16, 16)]
                    if crosses:
                        cv = c + lane
                        rowv = r5 + t + lax.shift_right_logical(cv, 7)
                        plsc.store_scatter(
                            obuf, [rowv, lax.bitwise_and(cv, 127)], v
                        )
                    else:
                        obuf[r5 + t, pl.ds(c, 16)] = v
                return cc

            lax.fori_loop(0, CHUNK, copy_body, 0)

        for par in range(2):
            @pl.when(p == par)
            def _(par=par):
                pltpu.async_copy(
                    obuf,
                    out_hbm.at[pl.ds((base + k * CHUNK) * NT, CHUNK * NT)],
                    wsems[par],
                )
        return carry

    lax.fori_loop(0, N_CHUNK, chunk_body, 0)

    for par in range(2):
        pltpu.make_async_copy(
            outbuf_v.at[par], out_hbm.at[pl.ds(base * NT, CHUNK * NT)], wsems[par]
        ).wait()


def kernel(x, emb):
    padded = _card_emb(x, emb)
    return padded.reshape(B, PAD_D)[:, :OUT_D]
